# Initial kernel scaffold; baseline (speedup 1.0000x reference)
import jax
import jax.numpy as jnp
from jax.experimental import pallas as pl


def kernel(vol, df):
    raise NotImplementedError("write your pallas kernel here")


# speedup vs baseline: 1.7248x; 1.7248x over previous
import functools

import jax
import jax.numpy as jnp
from jax import lax
from jax.experimental import pallas as pl
from jax.experimental.pallas import tpu as pltpu
from jax.experimental.pallas import tpu_sc as plsc

B = 2
C = 2
D = H = W = 128
DHW = D * H * W
NVOX = B * DHW
NW = 32
PER_W = NVOX // NW
CH = 2048
NCHUNK = PER_W // CH
VEC = 16
NIT = CH // VEC


def _warp(vol0, vol1, dff):
    mesh = plsc.VectorSubcoreMesh(core_axis_name="c", subcore_axis_name="s")

    scratch = (
        [pltpu.VMEM((CH,), jnp.float32) for _ in range(6)]
        + [pltpu.VMEM((CH,), jnp.int32) for _ in range(16)]
        + [pltpu.VMEM((CH,), jnp.float32) for _ in range(32)]
        + [pltpu.VMEM((CH,), jnp.float32) for _ in range(2)]
        + [pltpu.SemaphoreType.DMA, pltpu.SemaphoreType.DMA]
    )

    @functools.partial(
        pl.kernel,
        mesh=mesh,
        out_type=jax.ShapeDtypeStruct((B * C * DHW,), jnp.float32),
        scratch_types=scratch,
    )
    def k(v0h, v1h, dfh, outh, *s):
        fbuf = (s[0:3], s[3:6])
        ibuf = (s[6:14], s[14:22])
        g0buf = (s[22:30], s[38:46])
        g1buf = (s[30:38], s[46:54])
        o0, o1 = s[54], s[55]
        sems = (s[56], s[57])

        wid = lax.axis_index("s") * 2 + lax.axis_index("c")

        def prep(kc, p):
            gbase = wid * PER_W + kc * CH
            b = gbase >> 21
            sp0 = gbase & (DHW - 1)
            f0, f1, f2 = fbuf[p]
            idxs = ibuf[p]
            for d, fr in enumerate((f0, f1, f2)):
                off = pl.multiple_of((b * 3 + d) * DHW + sp0, CH)
                pltpu.sync_copy(dfh.at[pl.ds(off, CH)], fr)
            bb = b << 21

            def l1(i, c2):
                o = i * VEC
                vid = sp0 + o + lax.iota(jnp.int32, 16)
                z = vid >> 14
                y = (vid >> 7) & 127
                x = vid & 127

                dfz = f0[pl.ds(o, VEC)]
                locz = jnp.clip(z.astype(jnp.float32) + dfz, 0.0, 127.0)
                zb = jnp.minimum(locz.astype(jnp.int32), 126)
                f0[pl.ds(o, VEC)] = locz - zb.astype(jnp.float32)

                dfy = f1[pl.ds(o, VEC)]
                locy = jnp.clip(y.astype(jnp.float32) + dfy, 0.0, 127.0)
                yb = jnp.minimum(locy.astype(jnp.int32), 126)
                f1[pl.ds(o, VEC)] = locy - yb.astype(jnp.float32)

                dfx = f2[pl.ds(o, VEC)]
                locx = jnp.clip(x.astype(jnp.float32) + dfx, 0.0, 127.0)
                xb = jnp.minimum(locx.astype(jnp.int32), 126)
                f2[pl.ds(o, VEC)] = locx - xb.astype(jnp.float32)

                base = bb + (zb << 14) + (yb << 7) + xb
                for t, doff in enumerate((0, 1, 128, 129, 16384, 16385,
                                          16512, 16513)):
                    idxs[t][pl.ds(o, VEC)] = base + doff
                return c2

            lax.fori_loop(0, NIT, l1, 0)

        def fire(p):
            handles = []
            for ir, gr in zip(ibuf[p], g0buf[p]):
                handles.append(pltpu.async_copy(v0h.at[ir], gr, sems[p]))
            for ir, gr in zip(ibuf[p], g1buf[p]):
                handles.append(pltpu.async_copy(v1h.at[ir], gr, sems[p]))
            return handles

        def finish(kc, p, handles):
            for h in handles:
                h.wait()
            gbase = wid * PER_W + kc * CH
            b = gbase >> 21
            sp0 = gbase & (DHW - 1)
            f0, f1, f2 = fbuf[p]
            g0s, g1s = g0buf[p], g1buf[p]

            def l2(i, c2):
                o = i * VEC
                fz = f0[pl.ds(o, VEC)]
                fy = f1[pl.ds(o, VEC)]
                fx = f2[pl.ds(o, VEC)]
                gz = 1.0 - fz
                gy = 1.0 - fy
                gx = 1.0 - fx
                w00 = gz * gy
                w01 = gz * fy
                w10 = fz * gy
                w11 = fz * fy
                w = (w00 * gx, w00 * fx, w01 * gx, w01 * fx,
                     w10 * gx, w10 * fx, w11 * gx, w11 * fx)
                a0 = w[0] * g0s[0][pl.ds(o, VEC)]
                a1 = w[0] * g1s[0][pl.ds(o, VEC)]
                for t in range(1, 8):
                    a0 = a0 + w[t] * g0s[t][pl.ds(o, VEC)]
                    a1 = a1 + w[t] * g1s[t][pl.ds(o, VEC)]
                o0[pl.ds(o, VEC)] = a0
                o1[pl.ds(o, VEC)] = a1
                return c2

            lax.fori_loop(0, NIT, l2, 0)
            off0 = pl.multiple_of((b << 22) + sp0, CH)
            pltpu.sync_copy(o0, outh.at[pl.ds(off0, CH)])
            pltpu.sync_copy(o1, outh.at[pl.ds(pl.multiple_of(off0 + DHW, CH),
                                              CH)])

        prep(0, 0)

        def body(j, carry):
            for p in (0, 1):
                kc = 2 * j + p
                handles = fire(p)

                @pl.when(kc + 1 < NCHUNK)
                def _():
                    prep(kc + 1, 1 - p)

                finish(kc, p, handles)
            return carry

        lax.fori_loop(0, NCHUNK // 2, body, 0)

    return k(vol0, vol1, dff)


def kernel(vol, df):
    vol0 = vol[:, 0].reshape(-1)
    vol1 = vol[:, 1].reshape(-1)
    out = _warp(vol0, vol1, df.reshape(-1))
    return out.reshape(B, C, D, H, W)


# speedup vs baseline: 5.3657x; 3.1109x over previous
import functools

import jax
import jax.numpy as jnp
from jax import lax
from jax.experimental import pallas as pl
from jax.experimental.pallas import tpu as pltpu
from jax.experimental.pallas import tpu_sc as plsc

B = 2
C = 2
D = H = W = 128
DHW = D * H * W
NVOX = B * DHW
NW = 32
PER_W = NVOX // NW
CH = 2048
NCHUNK = PER_W // CH
VEC = 16
NIT = CH // VEC
SUB = 128
NSUB = CH // SUB
SIT = SUB // VEC


def _warp(tt, dff):
    mesh = plsc.VectorSubcoreMesh(core_axis_name="c", subcore_axis_name="s")

    scratch = (
        [pltpu.VMEM((CH,), jnp.float32) for _ in range(6)]
        + [pltpu.VMEM((CH,), jnp.int32) for _ in range(2)]
        + [pltpu.VMEM((SUB, 8), jnp.float32) for _ in range(2)]
        + [pltpu.VMEM((CH,), jnp.float32) for _ in range(4)]
        + [pltpu.SemaphoreType.DMA for _ in range(6)]
    )

    @functools.partial(
        pl.kernel,
        mesh=mesh,
        compiler_params=pltpu.CompilerParams(
            use_tc_tiling_on_sc=False, needs_layout_passes=False),
        out_type=jax.ShapeDtypeStruct((B * C * DHW,), jnp.float32),
        scratch_types=scratch,
    )
    def k(tth, dfh, outh, fa0, fa1, fa2, fb0, fb1, fb2, ia, ib,
          la, lb, oa0, oa1, ob0, ob1,
          sga, sgb, sda, sdb, soa, sob):
        fbuf = ((fa0, fa1, fa2), (fb0, fb1, fb2))
        ibuf = (ia, ib)
        land = (la, lb)
        obuf = ((oa0, oa1), (ob0, ob1))
        gsem = (sga, sgb)
        dsem = (sda, sdb)
        osem = (soa, sob)
        wid = lax.axis_index("s") * 2 + lax.axis_index("c")
        lane = lax.iota(jnp.int32, 16)
        cols = [jnp.full((16,), t, jnp.int32) for t in range(8)]

        def df_copies(kc, p):
            gbase = wid * PER_W + kc * CH
            b = gbase >> 21
            sp0 = gbase & (DHW - 1)
            out = []
            for d, fr in enumerate(fbuf[p]):
                off = pl.multiple_of((b * 3 + d) * DHW + sp0, CH)
                out.append(pltpu.make_async_copy(
                    dfh.at[pl.ds(off, CH)], fr, dsem[p]))
            return out

        def fire_df(kc, p):
            for cp in df_copies(kc, p):
                cp.start()

        def wait_df(kc, p):
            for cp in df_copies(kc, p):
                cp.wait()

        def gather_copies(kc, s, p):
            r = s & 1
            iv = ibuf[p].at[pl.ds(s * SUB, SUB)]
            return (pltpu.make_async_copy(tth.at[iv], land[r], gsem[r]),)

        def fire_sub(kc, s, p):
            for cp in gather_copies(kc, s, p):
                cp.start()

        def wait_sub(kc, s, p):
            for cp in gather_copies(kc, s, p):
                cp.wait()

        def out_copies(kc, p):
            gbase = wid * PER_W + kc * CH
            b = gbase >> 21
            sp0 = gbase & (DHW - 1)
            off0 = pl.multiple_of((b << 22) + sp0, CH)
            off1 = pl.multiple_of(off0 + DHW, CH)
            o0, o1 = obuf[p]
            return (pltpu.make_async_copy(o0, outh.at[pl.ds(off0, CH)],
                                          osem[p]),
                    pltpu.make_async_copy(o1, outh.at[pl.ds(off1, CH)],
                                          osem[p]))

        def loop1(kc, p):
            gbase = wid * PER_W + kc * CH
            b = gbase >> 21
            sp0 = gbase & (DHW - 1)
            bb = b << 21
            f0, f1, f2 = fbuf[p]
            idx = ibuf[p]

            def l1(i, c2):
                o = i * VEC
                vid = sp0 + o + lane
                z = vid >> 14
                y = (vid >> 7) & 127
                x = vid & 127

                dfz = f0[pl.ds(o, VEC)]
                locz = jnp.clip(z.astype(jnp.float32) + dfz, 0.0, 127.0)
                zb = jnp.minimum(locz.astype(jnp.int32), 126)
                f0[pl.ds(o, VEC)] = locz - zb.astype(jnp.float32)

                dfy = f1[pl.ds(o, VEC)]
                locy = jnp.clip(y.astype(jnp.float32) + dfy, 0.0, 127.0)
                yb = jnp.minimum(locy.astype(jnp.int32), 126)
                f1[pl.ds(o, VEC)] = locy - yb.astype(jnp.float32)

                dfx = f2[pl.ds(o, VEC)]
                locx = jnp.clip(x.astype(jnp.float32) + dfx, 0.0, 127.0)
                xb = jnp.minimum(locx.astype(jnp.int32), 126)
                f2[pl.ds(o, VEC)] = locx - xb.astype(jnp.float32)

                idx[pl.ds(o, VEC)] = bb + (zb << 14) + (yb << 7) + xb
                return c2

            lax.fori_loop(0, NIT, l1, 0)

        def sub_compute(s, p):
            r = s & 1
            lr = land[r]
            f0, f1, f2 = fbuf[p]
            o0, o1 = obuf[p]

            def l2(i, c2):
                o = s * SUB + i * VEC
                rvec = lane + i * VEC
                fz = f0[pl.ds(o, VEC)]
                fy = f1[pl.ds(o, VEC)]
                fx = f2[pl.ds(o, VEC)]
                gz = 1.0 - fz
                gy = 1.0 - fy
                gx = 1.0 - fx
                w00 = gz * gy
                w01 = gz * fy
                w10 = fz * gy
                w11 = fz * fy
                w = (w00 * gx, w00 * fx, w01 * gx, w01 * fx,
                     w10 * gx, w10 * fx, w11 * gx, w11 * fx)
                a0 = None
                a1 = None
                for j in range(4):
                    pv0 = plsc.load_gather(lr, [rvec, cols[j]])
                    lo0, hi0 = plsc.unpack(
                        plsc.bitcast(pv0, jnp.bfloat16),
                        format=plsc.PackFormat.INTERLEAVED,
                        preferred_element_type=jnp.float32)
                    pv1 = plsc.load_gather(lr, [rvec, cols[4 + j]])
                    lo1, hi1 = plsc.unpack(
                        plsc.bitcast(pv1, jnp.bfloat16),
                        format=plsc.PackFormat.INTERLEAVED,
                        preferred_element_type=jnp.float32)
                    t0_ = w[2 * j] * lo0 + w[2 * j + 1] * hi0
                    t1_ = w[2 * j] * lo1 + w[2 * j + 1] * hi1
                    a0 = t0_ if a0 is None else a0 + t0_
                    a1 = t1_ if a1 is None else a1 + t1_
                o0[pl.ds(o, VEC)] = a0
                o1[pl.ds(o, VEC)] = a1
                return c2

            lax.fori_loop(0, SIT, l2, 0)

        def chunk_body(kc, p):
            @pl.when(kc + 1 < NCHUNK)
            def _():
                fire_df(kc + 1, 1 - p)

            @pl.when(kc >= 2)
            def _():
                for cp in out_copies(kc - 2, p):
                    cp.wait()

            for s in range(NSUB):
                if s + 1 < NSUB:
                    fire_sub(kc, s + 1, p)
                if s == NSUB - 2:
                    @pl.when(kc + 1 < NCHUNK)
                    def _():
                        wait_df(kc + 1, 1 - p)
                        loop1(kc + 1, 1 - p)
                wait_sub(kc, s, p)
                sub_compute(s, p)

            @pl.when(kc + 1 < NCHUNK)
            def _():
                fire_sub(kc + 1, 0, 1 - p)

            for cp in out_copies(kc, p):
                cp.start()

        for cp in df_copies(0, 0):
            cp.start()
        for cp in df_copies(0, 0):
            cp.wait()
        loop1(0, 0)
        fire_sub(0, 0, 0)

        def body(j, carry):
            chunk_body(2 * j, 0)
            chunk_body(2 * j + 1, 1)
            return carry

        lax.fori_loop(0, NCHUNK // 2, body, 0)
        for cp in out_copies(NCHUNK - 2, 0):
            cp.wait()
        for cp in out_copies(NCHUNK - 1, 1):
            cp.wait()

    return k(tt, dff)


NZT = B * D // NW


def _build(volf):
    mesh = plsc.VectorSubcoreMesh(core_axis_name="c", subcore_axis_name="s")
    PW = H * W

    scratch = (
        [pltpu.VMEM((PW + VEC,), jnp.float32) for _ in range(4)]
        + [pltpu.VMEM((W, 8), jnp.float32) for _ in range(2)]
        + [pltpu.SemaphoreType.DMA, pltpu.SemaphoreType.DMA]
    )

    @functools.partial(
        pl.kernel,
        mesh=mesh,
        compiler_params=pltpu.CompilerParams(
            use_tc_tiling_on_sc=False, needs_layout_passes=False),
        out_type=jax.ShapeDtypeStruct((B * DHW, 8), jnp.float32),
        scratch_types=scratch,
    )
    def bk(vh, th, p00, p01, p10, p11, ta, tb, sp, so):
        planes = ((p00, p01), (p10, p11))
        otile = (ta, tb)
        wid = lax.axis_index("s") * 2 + lax.axis_index("c")
        lane = lax.iota(jnp.int32, 16)
        cols = [jnp.full((16,), t, jnp.int32) for t in range(8)]

        def ztile(t, carry):
            tid = wid * NZT + t
            b = tid >> 7
            z = tid & 127
            for c in (0, 1):
                for dz in (0, 1):
                    zs = jnp.minimum(z + dz, 127)
                    off = pl.multiple_of(((b * 2 + c) * 128 + zs) * PW, PW)
                    pltpu.sync_copy(vh.at[pl.ds(off, PW)],
                                    planes[c][dz].at[pl.ds(0, PW)])

            def yrow(y, r, cr):
                ot = otile[r]

                @pl.when(y >= 2)
                def _():
                    rb = pl.multiple_of(
                        (b << 21) + (z << 14) + ((y - 2) << 7), W)
                    pltpu.make_async_copy(ot, th.at[pl.ds(rb, W)], so).wait()

                def xv(i, c2):
                    o = i * VEC
                    for c in (0, 1):
                        for dz in (0, 1):
                            pl_ = planes[c][dz]
                            for dy in (0, 1):
                                ys = jnp.minimum(y + dy, 127)
                                lo = ys * W + o
                                a = pl_[pl.ds(lo, VEC)]
                                bv = pl_[pl.ds(lo + 1, VEC)]
                                wj = plsc.bitcast(
                                    plsc.pack(
                                        a, bv,
                                        format=plsc.PackFormat.INTERLEAVED),
                                    jnp.float32)
                                j = c * 4 + dz * 2 + dy
                                plsc.store_scatter(
                                    ot, [lane + o, cols[j]], wj)
                    return c2

                lax.fori_loop(0, W // VEC, xv, 0)
                rb = pl.multiple_of((b << 21) + (z << 14) + (y << 7), W)
                pltpu.make_async_copy(ot, th.at[pl.ds(rb, W)], so).start()
                return cr

            def ypair(jy, cr):
                yrow(2 * jy, 0, cr)
                yrow(2 * jy + 1, 1, cr)
                return cr

            lax.fori_loop(0, H // 2, ypair, 0)
            for yy in (H - 2, H - 1):
                rb = pl.multiple_of((b << 21) + (z << 14) + (yy << 7), W)
                pltpu.make_async_copy(otile[yy & 1], th.at[pl.ds(rb, W)],
                                      so).wait()
            return carry

        lax.fori_loop(0, NZT, ztile, 0)

    return bk(volf)


def kernel(vol, df):
    tt = _build(vol.reshape(-1))
    out = _warp(tt, df.reshape(-1))
    return out.reshape(B, C, D, H, W)
